# Initial kernel scaffold; baseline (speedup 1.0000x reference)
import jax
import jax.numpy as jnp
from jax.experimental import pallas as pl


def kernel(features, rbf_expansion, neighbor_list, W0, W1, b1, W2, b2, W3, b3, W4, b4):
    raise NotImplementedError("write your pallas kernel here")


# speedup vs baseline: 1.0001x; 1.0001x over previous
import functools

import jax
import jax.numpy as jnp
import numpy as np
from jax import lax
from jax.experimental import pallas as pl
from jax.experimental.pallas import tpu as pltpu
from jax.experimental.pallas import tpu_sc as plsc

_LOG2 = float(np.log(2.0))

_B, _N, _K, _G, _F = 8, 1024, 32, 64, 128
_ROWS = _B * _N
_EROWS = _ROWS * _K

_NC, _NS = 2, 16
_NW = _NC * _NS
_ITEMS_W = _ROWS // _NW
_CHUNK_ITEMS = 4
_CHUNK_IDX = _CHUNK_ITEMS * _K
_NCHUNKS = _ITEMS_W // _CHUNK_ITEMS


def _ssp(x):
    return jax.nn.softplus(x) - _LOG2


def _tc1_body(feat_ref, rbf_ref, w0_ref, w1_ref, b1_ref, w2_ref, b2_ref,
              init_ref, filt_ref):
    f = feat_ref[...].astype(jnp.bfloat16)
    init_ref[...] = jnp.dot(f, w0_ref[...], preferred_element_type=jnp.float32)
    x = rbf_ref[...].astype(jnp.bfloat16)
    x1 = _ssp(jnp.dot(x, w1_ref[...], preferred_element_type=jnp.float32)
              + b1_ref[...])
    filt_ref[...] = (jnp.dot(x1.astype(jnp.bfloat16), w2_ref[...],
                             preferred_element_type=jnp.float32)
                     + b2_ref[...])


def _tc1(feat2, rbf2, w0t, w1t, b1r, w2t, b2r):
    nblk = 64
    rows_blk = _ROWS // nblk
    erows_blk = _EROWS // nblk
    return pl.pallas_call(
        _tc1_body,
        grid=(nblk,),
        in_specs=[
            pl.BlockSpec((rows_blk, _F), lambda g: (g, 0)),
            pl.BlockSpec((erows_blk, _G), lambda g: (g, 0)),
            pl.BlockSpec((_F, _F), lambda g: (0, 0)),
            pl.BlockSpec((_G, _F), lambda g: (0, 0)),
            pl.BlockSpec((1, _F), lambda g: (0, 0)),
            pl.BlockSpec((_F, _F), lambda g: (0, 0)),
            pl.BlockSpec((1, _F), lambda g: (0, 0)),
        ],
        out_specs=[
            pl.BlockSpec((rows_blk, _F), lambda g: (g, 0)),
            pl.BlockSpec((erows_blk, _F), lambda g: (g, 0)),
        ],
        out_shape=[
            jax.ShapeDtypeStruct((_ROWS, _F), jnp.float32),
            jax.ShapeDtypeStruct((_EROWS, _F), jnp.float32),
        ],
    )(feat2, rbf2, w0t, w1t, b1r, w2t, b2r)


def _sc_body(init_hbm, filt_hbm, idx_hbm, agg_hbm,
             idx_v, gbuf, fbuf, obuf, sem_g, sem_f):
    wid = lax.axis_index("s") * _NC + lax.axis_index("c")
    idx_base = wid * (_ITEMS_W * _K)
    pltpu.sync_copy(idx_hbm.at[pl.ds(idx_base, _ITEMS_W * _K)], idx_v)

    def chunk(c, carry):
        gcp = pltpu.async_copy(
            init_hbm.at[idx_v.at[pl.ds(c * _CHUNK_IDX, _CHUNK_IDX)]],
            gbuf, sem_g)
        fcp = pltpu.async_copy(
            filt_hbm.at[pl.ds(idx_base + c * _CHUNK_IDX, _CHUNK_IDX)],
            fbuf, sem_f)
        gcp.wait()
        fcp.wait()
        for i in range(_CHUNK_ITEMS):
            base = i * _K
            for j in range(_F // 16):
                sl = pl.ds(j * 16, 16)
                acc = gbuf[base, sl] * fbuf[base, sl]
                for k in range(1, _K):
                    acc = acc + gbuf[base + k, sl] * fbuf[base + k, sl]
                obuf[pl.ds(c * (_CHUNK_ITEMS * _F) + i * _F + j * 16, 16)] = acc
        return carry

    lax.fori_loop(0, _NCHUNKS, chunk, 0)
    pltpu.sync_copy(obuf, agg_hbm.at[pl.ds(wid * (_ITEMS_W * _F),
                                           _ITEMS_W * _F)])


def _sc_conv(init_flat, filt_flat, idx_full):
    mesh = plsc.VectorSubcoreMesh(core_axis_name="c", subcore_axis_name="s")
    kern = functools.partial(
        pl.kernel,
        out_type=jax.ShapeDtypeStruct((_ROWS * _F,), jnp.float32),
        mesh=mesh,
        scratch_types=[
            pltpu.VMEM((_ITEMS_W * _K,), jnp.int32),
            pltpu.VMEM((_CHUNK_IDX, _F), jnp.float32),
            pltpu.VMEM((_CHUNK_IDX, _F), jnp.float32),
            pltpu.VMEM((_ITEMS_W * _F,), jnp.float32),
            pltpu.SemaphoreType.DMA,
            pltpu.SemaphoreType.DMA,
        ],
    )(_sc_body)
    return kern(init_flat, filt_flat, idx_full)


def _tc2_body(agg_ref, w3_ref, b3_ref, w4_ref, b4_ref, out_ref):
    z = _ssp(jnp.dot(agg_ref[...].astype(jnp.bfloat16), w3_ref[...],
                     preferred_element_type=jnp.float32) + b3_ref[...])
    out_ref[...] = (jnp.dot(z.astype(jnp.bfloat16), w4_ref[...],
                            preferred_element_type=jnp.float32) + b4_ref[...])


def _tc2(agg2, w3t, b3r, w4t, b4r):
    nblk = 8
    rows_blk = _ROWS // nblk
    return pl.pallas_call(
        _tc2_body,
        grid=(nblk,),
        in_specs=[
            pl.BlockSpec((rows_blk, _F), lambda g: (g, 0)),
            pl.BlockSpec((_F, _F), lambda g: (0, 0)),
            pl.BlockSpec((1, _F), lambda g: (0, 0)),
            pl.BlockSpec((_F, _F), lambda g: (0, 0)),
            pl.BlockSpec((1, _F), lambda g: (0, 0)),
        ],
        out_specs=pl.BlockSpec((rows_blk, _F), lambda g: (g, 0)),
        out_shape=jax.ShapeDtypeStruct((_ROWS, _F), jnp.float32),
    )(agg2, w3t, b3r, w4t, b4r)


def kernel(features, rbf_expansion, neighbor_list, W0, W1, b1, W2, b2,
           W3, b3, W4, b4):
    B, N, F = features.shape
    K = neighbor_list.shape[1]
    G = rbf_expansion.shape[-1]
    del G

    feat2 = features.reshape(B * N, F)
    rbf2 = rbf_expansion.reshape(B * N * K, _G)
    idx_full = (jnp.arange(B, dtype=jnp.int32)[:, None, None] * N
                + neighbor_list[None]).reshape(B * N * K)

    bf = jnp.bfloat16
    init_flat, filt_flat = _tc1(
        feat2, rbf2,
        W0.T.astype(bf), W1.T.astype(bf), b1.reshape(1, F),
        W2.T.astype(bf), b2.reshape(1, F))

    agg_flat = _sc_conv(init_flat, filt_flat, idx_full)

    out2 = _tc2(agg_flat.reshape(B * N, F),
                W3.T.astype(bf), b3.reshape(1, F),
                W4.T.astype(bf), b4.reshape(1, F))
    return out2.reshape(B, N, F)


# speedup vs baseline: 1.2025x; 1.2024x over previous
import functools

import jax
import jax.numpy as jnp
import numpy as np
from jax import lax
from jax.experimental import pallas as pl
from jax.experimental.pallas import tpu as pltpu
from jax.experimental.pallas import tpu_sc as plsc

_LOG2 = float(np.log(2.0))

_B, _N, _K, _G, _F = 8, 1024, 32, 64, 128
_ROWS = _B * _N
_EROWS = _ROWS * _K

_NC, _NS = 2, 16
_NW = _NC * _NS
_ITEMS_W = _ROWS // _NW
_CHUNK_ITEMS = 4
_CHUNK_IDX = _CHUNK_ITEMS * _K
_NCHUNKS = _ITEMS_W // _CHUNK_ITEMS


def _ssp(x):
    return jax.nn.softplus(x) - _LOG2


def _tc1_body(feat_ref, rbf_ref, w0_ref, w1_ref, b1_ref, w2_ref, b2_ref,
              init_ref, filt_ref):
    f = feat_ref[...].astype(jnp.bfloat16)
    init_ref[...] = jnp.dot(f, w0_ref[...], preferred_element_type=jnp.float32)
    x = rbf_ref[...].astype(jnp.bfloat16)
    x1 = _ssp(jnp.dot(x, w1_ref[...], preferred_element_type=jnp.float32)
              + b1_ref[...])
    filt_ref[...] = (jnp.dot(x1.astype(jnp.bfloat16), w2_ref[...],
                             preferred_element_type=jnp.float32)
                     + b2_ref[...])


def _tc1(feat2, rbf2, w0t, w1t, b1r, w2t, b2r):
    nblk = 64
    rows_blk = _ROWS // nblk
    erows_blk = _EROWS // nblk
    return pl.pallas_call(
        _tc1_body,
        grid=(nblk,),
        in_specs=[
            pl.BlockSpec((rows_blk, _F), lambda g: (g, 0)),
            pl.BlockSpec((erows_blk, _G), lambda g: (g, 0)),
            pl.BlockSpec((_F, _F), lambda g: (0, 0)),
            pl.BlockSpec((_G, _F), lambda g: (0, 0)),
            pl.BlockSpec((1, _F), lambda g: (0, 0)),
            pl.BlockSpec((_F, _F), lambda g: (0, 0)),
            pl.BlockSpec((1, _F), lambda g: (0, 0)),
        ],
        out_specs=[
            pl.BlockSpec((rows_blk, _F), lambda g: (g, 0)),
            pl.BlockSpec((erows_blk, _F), lambda g: (g, 0)),
        ],
        out_shape=[
            jax.ShapeDtypeStruct((_ROWS, _F), jnp.float32),
            jax.ShapeDtypeStruct((_EROWS, _F), jnp.float32),
        ],
    )(feat2, rbf2, w0t, w1t, b1r, w2t, b2r)


_WPB = _NW // _B


def _sc_body(init_hbm, filt_hbm, nbr_hbm, agg_hbm,
             idx_v, g0, g1, f0, f1, obuf,
             sem_g0, sem_g1, sem_f0, sem_f1):
    wid = lax.axis_index("s") * _NC + lax.axis_index("c")
    bn0 = (wid // _WPB) * _N
    nsl = (wid % _WPB) * (_ITEMS_W * _K)
    pltpu.sync_copy(nbr_hbm.at[pl.ds(nsl, _ITEMS_W * _K)], idx_v)
    bnv = jnp.full((16,), bn0, jnp.int32)

    def addb(i, carry):
        sl = pl.ds(i * 16, 16)
        idx_v[sl] = idx_v[sl] + bnv
        return carry

    lax.fori_loop(0, _ITEMS_W * _K // 16, addb, 0)

    gbufs, fbufs = (g0, g1), (f0, f1)
    gsems, fsems = (sem_g0, sem_g1), (sem_f0, sem_f1)
    frow0 = wid * (_ITEMS_W * _K)

    def issue(c, p):
        cw = lax.rem(c, _NCHUNKS)
        pltpu.async_copy(
            init_hbm.at[idx_v.at[pl.ds(cw * _CHUNK_IDX, _CHUNK_IDX)]],
            gbufs[p], gsems[p])
        pltpu.async_copy(
            filt_hbm.at[pl.ds(frow0 + cw * _CHUNK_IDX, _CHUNK_IDX)],
            fbufs[p], fsems[p])

    def wait(p):
        pltpu.make_async_copy(init_hbm.at[pl.ds(0, _CHUNK_IDX)],
                              gbufs[p], gsems[p]).wait()
        pltpu.make_async_copy(filt_hbm.at[pl.ds(0, _CHUNK_IDX)],
                              fbufs[p], fsems[p]).wait()

    issue(0, 0)
    issue(1, 1)

    def pair(c2, carry):
        for p in range(2):
            c = c2 * 2 + p
            wait(p)
            gbuf, fbuf = gbufs[p], fbufs[p]
            for i in range(_CHUNK_ITEMS):
                base = i * _K
                for j in range(_F // 16):
                    sl = pl.ds(j * 16, 16)
                    acc = gbuf[base, sl] * fbuf[base, sl]
                    for k in range(1, _K):
                        acc = acc + gbuf[base + k, sl] * fbuf[base + k, sl]
                    obuf[pl.ds(c * (_CHUNK_ITEMS * _F) + i * _F + j * 16,
                               16)] = acc
            issue(c + 2, p)
        return carry

    lax.fori_loop(0, _NCHUNKS // 2, pair, 0)
    for p in range(2):
        wait(p)
    pltpu.sync_copy(obuf, agg_hbm.at[pl.ds(wid * (_ITEMS_W * _F),
                                           _ITEMS_W * _F)])


def _sc_conv(init_flat, filt_flat, nbr_flat):
    mesh = plsc.VectorSubcoreMesh(core_axis_name="c", subcore_axis_name="s")
    kern = functools.partial(
        pl.kernel,
        out_type=jax.ShapeDtypeStruct((_ROWS * _F,), jnp.float32),
        mesh=mesh,
        scratch_types=[
            pltpu.VMEM((_ITEMS_W * _K,), jnp.int32),
            pltpu.VMEM((_CHUNK_IDX, _F), jnp.float32),
            pltpu.VMEM((_CHUNK_IDX, _F), jnp.float32),
            pltpu.VMEM((_CHUNK_IDX, _F), jnp.float32),
            pltpu.VMEM((_CHUNK_IDX, _F), jnp.float32),
            pltpu.VMEM((_ITEMS_W * _F,), jnp.float32),
            pltpu.SemaphoreType.DMA,
            pltpu.SemaphoreType.DMA,
            pltpu.SemaphoreType.DMA,
            pltpu.SemaphoreType.DMA,
        ],
    )(_sc_body)
    return kern(init_flat, filt_flat, nbr_flat)


def _tc2_body(agg_ref, w3_ref, b3_ref, w4_ref, b4_ref, out_ref):
    z = _ssp(jnp.dot(agg_ref[...].astype(jnp.bfloat16), w3_ref[...],
                     preferred_element_type=jnp.float32) + b3_ref[...])
    out_ref[...] = (jnp.dot(z.astype(jnp.bfloat16), w4_ref[...],
                            preferred_element_type=jnp.float32) + b4_ref[...])


def _tc2(agg2, w3t, b3r, w4t, b4r):
    nblk = 8
    rows_blk = _ROWS // nblk
    return pl.pallas_call(
        _tc2_body,
        grid=(nblk,),
        in_specs=[
            pl.BlockSpec((rows_blk, _F), lambda g: (g, 0)),
            pl.BlockSpec((_F, _F), lambda g: (0, 0)),
            pl.BlockSpec((1, _F), lambda g: (0, 0)),
            pl.BlockSpec((_F, _F), lambda g: (0, 0)),
            pl.BlockSpec((1, _F), lambda g: (0, 0)),
        ],
        out_specs=pl.BlockSpec((rows_blk, _F), lambda g: (g, 0)),
        out_shape=jax.ShapeDtypeStruct((_ROWS, _F), jnp.float32),
    )(agg2, w3t, b3r, w4t, b4r)


def kernel(features, rbf_expansion, neighbor_list, W0, W1, b1, W2, b2,
           W3, b3, W4, b4):
    B, N, F = features.shape
    K = neighbor_list.shape[1]
    G = rbf_expansion.shape[-1]
    del G

    feat2 = features.reshape(B * N, F)
    rbf2 = rbf_expansion.reshape(B * N * K, _G)
    nbr_flat = neighbor_list.reshape(N * K)

    bf = jnp.bfloat16
    init_flat, filt_flat = _tc1(
        feat2, rbf2,
        W0.T.astype(bf), W1.T.astype(bf), b1.reshape(1, F),
        W2.T.astype(bf), b2.reshape(1, F))

    agg_flat = _sc_conv(init_flat, filt_flat, nbr_flat)

    out2 = _tc2(agg_flat.reshape(B * N, F),
                W3.T.astype(bf), b3.reshape(1, F),
                W4.T.astype(bf), b4.reshape(1, F))
    return out2.reshape(B, N, F)


# speedup vs baseline: 1.2082x; 1.0047x over previous
import functools

import jax
import jax.numpy as jnp
import numpy as np
from jax import lax
from jax.experimental import pallas as pl
from jax.experimental.pallas import tpu as pltpu
from jax.experimental.pallas import tpu_sc as plsc

_LOG2 = float(np.log(2.0))

_B, _N, _K, _G, _F = 8, 1024, 32, 64, 128
_ROWS = _B * _N
_EROWS = _ROWS * _K

_NC, _NS = 2, 16
_NW = _NC * _NS
_ITEMS_W = _ROWS // _NW
_CHUNK_ITEMS = 4
_CHUNK_IDX = _CHUNK_ITEMS * _K
_NCHUNKS = _ITEMS_W // _CHUNK_ITEMS


def _ssp(x):
    return jax.nn.softplus(x) - _LOG2


def _tc1_body(feat_ref, rbf_ref, w0_ref, w1_ref, b1_ref, w2_ref, b2_ref,
              init_ref, filt_ref):
    f = feat_ref[...].astype(jnp.bfloat16)
    init_ref[...] = jnp.dot(f, w0_ref[...], preferred_element_type=jnp.float32)
    x = rbf_ref[...].astype(jnp.bfloat16)
    x1 = _ssp(jnp.dot(x, w1_ref[...], preferred_element_type=jnp.float32)
              + b1_ref[...])
    filt_ref[...] = (jnp.dot(x1.astype(jnp.bfloat16), w2_ref[...],
                             preferred_element_type=jnp.float32)
                     + b2_ref[...])


def _tc1(feat2, rbf2, w0t, w1t, b1r, w2t, b2r):
    nblk = 64
    rows_blk = _ROWS // nblk
    erows_blk = _EROWS // nblk
    return pl.pallas_call(
        _tc1_body,
        grid=(nblk,),
        in_specs=[
            pl.BlockSpec((rows_blk, _F), lambda g: (g, 0)),
            pl.BlockSpec((erows_blk, _G), lambda g: (g, 0)),
            pl.BlockSpec((_F, _F), lambda g: (0, 0)),
            pl.BlockSpec((_G, _F), lambda g: (0, 0)),
            pl.BlockSpec((1, _F), lambda g: (0, 0)),
            pl.BlockSpec((_F, _F), lambda g: (0, 0)),
            pl.BlockSpec((1, _F), lambda g: (0, 0)),
        ],
        out_specs=[
            pl.BlockSpec((rows_blk, _F), lambda g: (g, 0)),
            pl.BlockSpec((erows_blk, _F), lambda g: (g, 0)),
        ],
        out_shape=[
            jax.ShapeDtypeStruct((_ROWS, _F), jnp.float32),
            jax.ShapeDtypeStruct((_EROWS, _F), jnp.float32),
        ],
    )(feat2, rbf2, w0t, w1t, b1r, w2t, b2r)


_WPB = _NW // _B


def _sc_body(init_hbm, filt_hbm, nbr_hbm, agg_hbm,
             idx_v, g0, g1, f0, f1, obuf,
             sem_g0, sem_g1, sem_f0, sem_f1):
    wid = lax.axis_index("s") * _NC + lax.axis_index("c")
    bn0 = (wid // _WPB) * _N
    nsl = (wid % _WPB) * (_ITEMS_W * _K)
    pltpu.sync_copy(nbr_hbm.at[pl.ds(nsl, _ITEMS_W * _K)], idx_v)
    bnv = jnp.full((16,), bn0, jnp.int32)

    def addb(i, carry):
        sl = pl.ds(i * 16, 16)
        idx_v[sl] = idx_v[sl] + bnv
        return carry

    lax.fori_loop(0, _ITEMS_W * _K // 16, addb, 0)

    gbufs, fbufs = (g0, g1), (f0, f1)
    gsems, fsems = (sem_g0, sem_g1), (sem_f0, sem_f1)
    frow0 = wid * (_ITEMS_W * _K)

    def issue(c, p):
        cw = lax.rem(c, _NCHUNKS)
        pltpu.async_copy(
            init_hbm.at[idx_v.at[pl.ds(cw * _CHUNK_IDX, _CHUNK_IDX)]],
            gbufs[p], gsems[p])
        pltpu.async_copy(
            filt_hbm.at[pl.ds(frow0 + cw * _CHUNK_IDX, _CHUNK_IDX)],
            fbufs[p], fsems[p])

    def wait(p):
        pltpu.make_async_copy(init_hbm.at[pl.ds(0, _CHUNK_IDX)],
                              gbufs[p], gsems[p]).wait()
        pltpu.make_async_copy(filt_hbm.at[pl.ds(0, _CHUNK_IDX)],
                              fbufs[p], fsems[p]).wait()

    issue(0, 0)
    issue(1, 1)

    def pair(c2, carry):
        for p in range(2):
            c = c2 * 2 + p
            wait(p)
            gbuf, fbuf = gbufs[p], fbufs[p]
            for i in range(_CHUNK_ITEMS):
                base = i * _K
                for j in range(_F // 16):
                    sl = pl.ds(j * 16, 16)
                    acc = gbuf[base, sl] * fbuf[base, sl]
                    for k in range(1, _K):
                        acc = acc + gbuf[base + k, sl] * fbuf[base + k, sl]
                    obuf[c * _CHUNK_ITEMS + i, sl] = acc
            issue(c + 2, p)
        return carry

    lax.fori_loop(0, _NCHUNKS // 2, pair, 0)
    for p in range(2):
        wait(p)
    pltpu.sync_copy(obuf, agg_hbm.at[pl.ds(wid * _ITEMS_W, _ITEMS_W)])


def _sc_conv(init_flat, filt_flat, nbr_flat):
    mesh = plsc.VectorSubcoreMesh(core_axis_name="c", subcore_axis_name="s")
    kern = functools.partial(
        pl.kernel,
        out_type=jax.ShapeDtypeStruct((_ROWS, _F), jnp.float32),
        mesh=mesh,
        scratch_types=[
            pltpu.VMEM((_ITEMS_W * _K,), jnp.int32),
            pltpu.VMEM((_CHUNK_IDX, _F), jnp.float32),
            pltpu.VMEM((_CHUNK_IDX, _F), jnp.float32),
            pltpu.VMEM((_CHUNK_IDX, _F), jnp.float32),
            pltpu.VMEM((_CHUNK_IDX, _F), jnp.float32),
            pltpu.VMEM((_ITEMS_W, _F), jnp.float32),
            pltpu.SemaphoreType.DMA,
            pltpu.SemaphoreType.DMA,
            pltpu.SemaphoreType.DMA,
            pltpu.SemaphoreType.DMA,
        ],
    )(_sc_body)
    return kern(init_flat, filt_flat, nbr_flat)


def _tc2_body(agg_ref, w3_ref, b3_ref, w4_ref, b4_ref, out_ref):
    z = _ssp(jnp.dot(agg_ref[...].astype(jnp.bfloat16), w3_ref[...],
                     preferred_element_type=jnp.float32) + b3_ref[...])
    out_ref[...] = (jnp.dot(z.astype(jnp.bfloat16), w4_ref[...],
                            preferred_element_type=jnp.float32) + b4_ref[...])


def _tc2(agg2, w3t, b3r, w4t, b4r):
    nblk = 8
    rows_blk = _ROWS // nblk
    return pl.pallas_call(
        _tc2_body,
        grid=(nblk,),
        in_specs=[
            pl.BlockSpec((rows_blk, _F), lambda g: (g, 0)),
            pl.BlockSpec((_F, _F), lambda g: (0, 0)),
            pl.BlockSpec((1, _F), lambda g: (0, 0)),
            pl.BlockSpec((_F, _F), lambda g: (0, 0)),
            pl.BlockSpec((1, _F), lambda g: (0, 0)),
        ],
        out_specs=pl.BlockSpec((rows_blk, _F), lambda g: (g, 0)),
        out_shape=jax.ShapeDtypeStruct((_ROWS, _F), jnp.float32),
    )(agg2, w3t, b3r, w4t, b4r)


def kernel(features, rbf_expansion, neighbor_list, W0, W1, b1, W2, b2,
           W3, b3, W4, b4):
    B, N, F = features.shape
    K = neighbor_list.shape[1]
    G = rbf_expansion.shape[-1]
    del G

    feat2 = features.reshape(B * N, F)
    rbf2 = rbf_expansion.reshape(B * N * K, _G)
    nbr_flat = neighbor_list.reshape(N * K)

    bf = jnp.bfloat16
    init_flat, filt_flat = _tc1(
        feat2, rbf2,
        W0.T.astype(bf), W1.T.astype(bf), b1.reshape(1, F),
        W2.T.astype(bf), b2.reshape(1, F))

    agg_flat = _sc_conv(init_flat, filt_flat, nbr_flat)

    out2 = _tc2(agg_flat,
                W3.T.astype(bf), b3.reshape(1, F),
                W4.T.astype(bf), b4.reshape(1, F))
    return out2.reshape(B, N, F)
